# baseline (speedup 1.0000x reference)
import functools

import jax
import jax.numpy as jnp
from jax import lax
from jax.experimental import pallas as pl
from jax.experimental.pallas import tpu as pltpu
from jax.experimental.pallas import tpu_sc as plsc

NUM_CORES = 2
NUM_SUBCORES = 16
NUM_WORKERS = NUM_CORES * NUM_SUBCORES
CHUNK = 1280

_MESH = dict(core_axis_name="c", subcore_axis_name="s")


def _wid():
    return lax.axis_index("s") * NUM_CORES + lax.axis_index("c")


def _transpose_block(blk, stag, nvg):
    iota = lax.iota(jnp.int32, 16)
    vls = [iota + 16 * vg for vg in range(nvg)]
    vl32s = [vl * 32 for vl in vls]

    def tbody(t2, carry):
        work = []
        for u in range(2):
            d_off = (iota + (2 * t2 + u)) & 15
            for dg in range(2):
                d_idx = d_off + 16 * dg
                for vg in range(nvg):
                    val = plsc.load_gather(blk, [d_idx, vls[vg]])
                    work.append((vl32s[vg] + d_idx, val))
        for addr, val in work:
            plsc.store_scatter(stag, [addr], val)
        return carry

    lax.fori_loop(0, 8, tbody, 0)


@jax.jit
def _untile(tab_t, tail_lin):
    d, v = tab_t.shape
    n_blk_full = v // 128
    v_tail = v - n_blk_full * 128
    n_main = (n_blk_full // NUM_WORKERS) & ~1
    n_rem = n_blk_full - n_main * NUM_WORKERS

    @functools.partial(
        pl.kernel,
        out_type=jax.ShapeDtypeStruct((v * d,), jnp.float32),
        mesh=plsc.VectorSubcoreMesh(**_MESH),
        scratch_types=[
            pltpu.VMEM((d, 128), jnp.float32),
            pltpu.VMEM((d, 128), jnp.float32),
            pltpu.VMEM((128 * d,), jnp.float32),
            pltpu.VMEM((128 * d,), jnp.float32),
            pltpu.SemaphoreType.DMA,
            pltpu.SemaphoreType.DMA,
            pltpu.SemaphoreType.DMA,
            pltpu.SemaphoreType.DMA,
        ],
        compiler_params=pltpu.CompilerParams(
            use_tc_tiling_on_sc=True, needs_layout_passes=False
        ),
    )
    def untile(t_hbm, tail_hbm, out_hbm, blk0, blk1, st0, st1, is0, is1, os0, os1):
        wid = _wid()
        blks = (blk0, blk1)
        stags = (st0, st1)
        isem = (is0, is1)
        osem = (os0, os1)

        def blk_idx(i):
            return wid + NUM_WORKERS * i

        def start_in(i, b):
            pltpu.async_copy(
                t_hbm.at[:, pl.ds(blk_idx(i) * 128, 128)], blks[b], isem[b]
            )

        def wait_in(i, b):
            pltpu.make_async_copy(
                t_hbm.at[:, pl.ds(blk_idx(i) * 128, 128)], blks[b], isem[b]
            ).wait()

        def start_out(i, b):
            pltpu.async_copy(
                stags[b], out_hbm.at[pl.ds(blk_idx(i) * 128 * d, 128 * d)], osem[b]
            )

        def wait_out(i, b):
            pltpu.make_async_copy(
                stags[b], out_hbm.at[pl.ds(blk_idx(i) * 128 * d, 128 * d)], osem[b]
            ).wait()

        start_in(0, 0)
        start_in(1, 1)

        def pair_body(p, carry):
            for b in (0, 1):
                i = 2 * p + b
                wait_in(i, b)

                @pl.when(i >= 2)
                def _():
                    wait_out(i - 2, b)

                _transpose_block(blks[b], stags[b], 8)
                start_out(i, b)

                @pl.when(i + 2 < n_main)
                def _():
                    start_in(i + 2, b)

            return carry

        lax.fori_loop(0, n_main // 2, pair_body, 0)
        wait_out(n_main - 2, 0)
        wait_out(n_main - 1, 1)

        @pl.when(wid < n_rem)
        def _():
            c = n_main * NUM_WORKERS + wid
            pltpu.sync_copy(t_hbm.at[:, pl.ds(c * 128, 128)], blk0)
            _transpose_block(blk0, st0, 8)
            pltpu.sync_copy(st0, out_hbm.at[pl.ds(c * 128 * d, 128 * d)])

        @pl.when(wid == n_rem)
        def _():
            base = n_blk_full * 128
            pltpu.sync_copy(tail_hbm, st0.at[pl.ds(0, v_tail * d)])
            pltpu.sync_copy(
                st0.at[pl.ds(0, v_tail * d)],
                out_hbm.at[pl.ds(base * d, v_tail * d)],
            )

    return untile(tab_t, tail_lin)


@jax.jit
def _embed_native(xt, table):
    ll, bb = xt.shape
    _, d = table.shape
    bpw = bb // NUM_WORKERS
    n_dtiles = d // 8

    @functools.partial(
        pl.kernel,
        out_type=jax.ShapeDtypeStruct((ll * d * bb,), jnp.float32),
        mesh=plsc.VectorSubcoreMesh(**_MESH),
        scratch_types=[
            pltpu.VMEM((ll, bpw), jnp.int32),
            pltpu.VMEM((bpw, d), jnp.float32),
            pltpu.VMEM((bpw, d), jnp.float32),
            pltpu.VMEM((bpw * d,), jnp.float32),
            pltpu.VMEM((bpw * d,), jnp.float32),
            pltpu.SemaphoreType.DMA,
            pltpu.SemaphoreType.DMA,
            pltpu.SemaphoreType.DMA,
            pltpu.SemaphoreType.DMA,
        ],
        compiler_params=pltpu.CompilerParams(
            use_tc_tiling_on_sc=False, needs_layout_passes=False
        ),
    )
    def emb(x_hbm, tab_hbm, out_hbm, idx_v, g0, g1, st0, st1, gs0, gs1, os0, os1):
        wid = _wid()
        gv = (g0, g1)
        stags = (st0, st1)
        gsem = (gs0, gs1)
        osem = (os0, os1)

        pltpu.sync_copy(x_hbm.at[:, pl.ds(wid * bpw, bpw)], idx_v)
        iota = lax.iota(jnp.int32, 16)

        def start_gather(l, b):
            pltpu.async_copy(tab_hbm.at[idx_v.at[l]], gv[b], gsem[b])

        def wait_gather(l, b):
            pltpu.make_async_copy(
                tab_hbm.at[idx_v.at[l]], gv[b], gsem[b]
            ).wait()

        def out_slice(l, a):
            return out_hbm.at[
                pl.ds(((l * n_dtiles + a) * d + wid) * bpw * 8, bpw * 8)
            ]

        def start_out(l, b):
            for a in range(n_dtiles):
                pltpu.async_copy(
                    stags[b].at[pl.ds(a * bpw * 8, bpw * 8)],
                    out_slice(l, a),
                    osem[b],
                )

        def wait_out(l, b):
            for a in range(n_dtiles):
                pltpu.make_async_copy(
                    stags[b].at[pl.ds(a * bpw * 8, bpw * 8)],
                    out_slice(l, a),
                    osem[b],
                ).wait()

        jjs = [iota + 16 * jg for jg in range(bpw // 16)]

        def transpose(b):
            def tbody(t2, carry):
                work = []
                for u in range(2):
                    d_off = (iota + (2 * t2 + u)) & 15
                    for dg in range(2):
                        dd = d_off + 16 * dg
                        dst_base = (dd >> 3) * (bpw * 8) + (dd & 7) * bpw
                        for jg in range(bpw // 16):
                            val = plsc.load_gather(gv[b], [jjs[jg], dd])
                            work.append((dst_base + jjs[jg], val))
                for addr, val in work:
                    plsc.store_scatter(stags[b], [addr], val)
                return carry

            lax.fori_loop(0, 8, tbody, 0)

        start_gather(0, 0)
        start_gather(1, 1)

        def pair_body(p, carry):
            for b in (0, 1):
                l = 2 * p + b
                wait_gather(l, b)

                @pl.when(l >= 2)
                def _():
                    wait_out(l - 2, b)

                transpose(b)
                start_out(l, b)

                @pl.when(l + 2 < ll)
                def _():
                    start_gather(l + 2, b)

            return carry

        lax.fori_loop(0, ll // 2, pair_body, 0)
        wait_out(ll - 2, 0)
        wait_out(ll - 1, 1)

    return emb(xt, table)


def kernel(x, table):
    b, l = x.shape
    v, d = table.shape
    n_full = (v // 128) * 128
    tail_lin = table[n_full:, :].reshape((v - n_full) * d)
    table_lin = _untile(table.T, tail_lin).reshape(v, d)
    out1d = _embed_native(x.T, table_lin)
    t5 = out1d.reshape(l, d // 8, b // 128, 8, 128)
    return t5.transpose(2, 4, 0, 1, 3).reshape(b, l, d)


# speedup vs baseline: 1.0931x; 1.0931x over previous
import functools

import jax
import jax.numpy as jnp
from jax import lax
from jax.experimental import pallas as pl
from jax.experimental.pallas import tpu as pltpu
from jax.experimental.pallas import tpu_sc as plsc

NUM_CORES = 2
NUM_SUBCORES = 16
NUM_WORKERS = NUM_CORES * NUM_SUBCORES
CHUNK = 1280

_MESH = dict(core_axis_name="c", subcore_axis_name="s")


def _wid():
    return lax.axis_index("s") * NUM_CORES + lax.axis_index("c")


def _transpose_block(blk, stag, nvg):
    iota = lax.iota(jnp.int32, 16)
    vls = [iota + 16 * vg for vg in range(nvg)]
    vl32s = [vl * 32 for vl in vls]

    def tbody(t, carry):
        d_off = (iota + t) & 15
        work = []
        for dg in range(2):
            d_idx = d_off + 16 * dg
            for vg in range(nvg):
                val = plsc.load_gather(blk, [d_idx, vls[vg]])
                work.append((vl32s[vg] + d_idx, val))
        for addr, val in work:
            plsc.store_scatter(stag, [addr], val)
        return carry

    lax.fori_loop(0, 16, tbody, 0)


@jax.jit
def _untile(tab_t, tail_lin):
    d, v = tab_t.shape
    w = 256
    v_tail = v - (v // 128) * 128
    n_blk_full = (v - v_tail) // w
    n_main = (n_blk_full // NUM_WORKERS) & ~1
    n_rem = n_blk_full - n_main * NUM_WORKERS

    @functools.partial(
        pl.kernel,
        out_type=jax.ShapeDtypeStruct((v * d,), jnp.float32),
        mesh=plsc.VectorSubcoreMesh(**_MESH),
        scratch_types=[
            pltpu.VMEM((d, w), jnp.float32),
            pltpu.VMEM((d, w), jnp.float32),
            pltpu.VMEM((w * d,), jnp.float32),
            pltpu.VMEM((w * d,), jnp.float32),
            pltpu.SemaphoreType.DMA,
            pltpu.SemaphoreType.DMA,
            pltpu.SemaphoreType.DMA,
            pltpu.SemaphoreType.DMA,
        ],
        compiler_params=pltpu.CompilerParams(
            use_tc_tiling_on_sc=True, needs_layout_passes=False
        ),
    )
    def untile(t_hbm, tail_hbm, out_hbm, blk0, blk1, st0, st1, is0, is1, os0, os1):
        wid = _wid()
        blks = (blk0, blk1)
        stags = (st0, st1)
        isem = (is0, is1)
        osem = (os0, os1)

        def blk_idx(i):
            return wid + NUM_WORKERS * i

        def start_in(i, b):
            pltpu.async_copy(
                t_hbm.at[:, pl.ds(blk_idx(i) * w, w)], blks[b], isem[b]
            )

        def wait_in(i, b):
            pltpu.make_async_copy(
                t_hbm.at[:, pl.ds(blk_idx(i) * w, w)], blks[b], isem[b]
            ).wait()

        def start_out(i, b):
            pltpu.async_copy(
                stags[b], out_hbm.at[pl.ds(blk_idx(i) * w * d, w * d)], osem[b]
            )

        def wait_out(i, b):
            pltpu.make_async_copy(
                stags[b], out_hbm.at[pl.ds(blk_idx(i) * w * d, w * d)], osem[b]
            ).wait()

        start_in(0, 0)
        start_in(1, 1)

        def pair_body(p, carry):
            for b in (0, 1):
                i = 2 * p + b
                wait_in(i, b)

                @pl.when(i >= 2)
                def _():
                    wait_out(i - 2, b)

                _transpose_block(blks[b], stags[b], w // 16)
                start_out(i, b)

                @pl.when(i + 2 < n_main)
                def _():
                    start_in(i + 2, b)

            return carry

        lax.fori_loop(0, n_main // 2, pair_body, 0)
        wait_out(n_main - 2, 0)
        wait_out(n_main - 1, 1)

        @pl.when(wid < n_rem)
        def _():
            c = n_main * NUM_WORKERS + wid
            pltpu.sync_copy(t_hbm.at[:, pl.ds(c * w, w)], blk0)
            _transpose_block(blk0, st0, w // 16)
            pltpu.sync_copy(st0, out_hbm.at[pl.ds(c * w * d, w * d)])

        @pl.when(wid == n_rem)
        def _():
            base = n_blk_full * w
            pltpu.sync_copy(tail_hbm, st0.at[pl.ds(0, v_tail * d)])
            pltpu.sync_copy(
                st0.at[pl.ds(0, v_tail * d)],
                out_hbm.at[pl.ds(base * d, v_tail * d)],
            )

    return untile(tab_t, tail_lin)


@jax.jit
def _embed_native(xt, table):
    ll, bb = xt.shape
    _, d = table.shape
    bpw = bb // NUM_WORKERS
    n_dtiles = d // 8

    @functools.partial(
        pl.kernel,
        out_type=jax.ShapeDtypeStruct((ll * d * bb,), jnp.float32),
        mesh=plsc.VectorSubcoreMesh(**_MESH),
        scratch_types=[
            pltpu.VMEM((ll, bpw), jnp.int32),
            pltpu.VMEM((bpw, d), jnp.float32),
            pltpu.VMEM((bpw, d), jnp.float32),
            pltpu.VMEM((bpw * d,), jnp.float32),
            pltpu.VMEM((bpw * d,), jnp.float32),
            pltpu.SemaphoreType.DMA,
            pltpu.SemaphoreType.DMA,
            pltpu.SemaphoreType.DMA,
            pltpu.SemaphoreType.DMA,
        ],
        compiler_params=pltpu.CompilerParams(
            use_tc_tiling_on_sc=False, needs_layout_passes=False
        ),
    )
    def emb(x_hbm, tab_hbm, out_hbm, idx_v, g0, g1, st0, st1, gs0, gs1, os0, os1):
        wid = _wid()
        gv = (g0, g1)
        stags = (st0, st1)
        gsem = (gs0, gs1)
        osem = (os0, os1)

        pltpu.sync_copy(x_hbm.at[:, pl.ds(wid * bpw, bpw)], idx_v)
        iota = lax.iota(jnp.int32, 16)

        def start_gather(l, b):
            pltpu.async_copy(tab_hbm.at[idx_v.at[l]], gv[b], gsem[b])

        def wait_gather(l, b):
            pltpu.make_async_copy(
                tab_hbm.at[idx_v.at[l]], gv[b], gsem[b]
            ).wait()

        def out_slice(l, a):
            return out_hbm.at[
                pl.ds(((l * n_dtiles + a) * d + wid) * bpw * 8, bpw * 8)
            ]

        def start_out(l, b):
            for a in range(n_dtiles):
                pltpu.async_copy(
                    stags[b].at[pl.ds(a * bpw * 8, bpw * 8)],
                    out_slice(l, a),
                    osem[b],
                )

        def wait_out(l, b):
            for a in range(n_dtiles):
                pltpu.make_async_copy(
                    stags[b].at[pl.ds(a * bpw * 8, bpw * 8)],
                    out_slice(l, a),
                    osem[b],
                ).wait()

        jjs = [iota + 16 * jg for jg in range(bpw // 16)]

        def transpose(b):
            def tbody(t, carry):
                d_off = (iota + t) & 15
                work = []
                for dg in range(2):
                    dd = d_off + 16 * dg
                    dst_base = (dd >> 3) * (bpw * 8) + (dd & 7) * bpw
                    for jg in range(bpw // 16):
                        val = plsc.load_gather(gv[b], [jjs[jg], dd])
                        work.append((dst_base + jjs[jg], val))
                for addr, val in work:
                    plsc.store_scatter(stags[b], [addr], val)
                return carry

            lax.fori_loop(0, 16, tbody, 0)

        start_gather(0, 0)
        start_gather(1, 1)

        def pair_body(p, carry):
            for b in (0, 1):
                l = 2 * p + b
                wait_gather(l, b)

                @pl.when(l >= 2)
                def _():
                    wait_out(l - 2, b)

                transpose(b)
                start_out(l, b)

                @pl.when(l + 2 < ll)
                def _():
                    start_gather(l + 2, b)

            return carry

        lax.fori_loop(0, ll // 2, pair_body, 0)
        wait_out(ll - 2, 0)
        wait_out(ll - 1, 1)

    return emb(xt, table)


def kernel(x, table):
    b, l = x.shape
    v, d = table.shape
    n_full = (v // 128) * 128
    tail_lin = table[n_full:, :].reshape((v - n_full) * d)
    table_lin = _untile(table.T, tail_lin).reshape(v, d)
    out1d = _embed_native(x.T, table_lin)
    t5 = out1d.reshape(l, d // 8, b // 128, 8, 128)
    return t5.transpose(2, 4, 0, 1, 3).reshape(b, l, d)
